# baseline (speedup 1.0000x reference)
import jax
import jax.numpy as jnp
from jax.experimental import pallas as pl
from jax.experimental.pallas import tpu as pltpu

N_BUCKETS = 512
S = 32
TAU = 0.1
ALPHA = 0.5
TB = 8


def _token_kernel(sb_ref, st_ref, order_ref,
                  *refs):
    q_refs = refs[0:TB]
    k_refs = refs[TB:2 * TB]
    v_refs = refs[2 * TB:3 * TB]
    st_refs = refs[3 * TB:4 * TB]
    cb_ref = refs[4 * TB]
    out_refs = refs[4 * TB + 1:5 * TB + 1]
    sim_refs = refs[5 * TB + 1:6 * TB + 1]

    i = pl.program_id(0)
    gtot = pl.num_programs(0)

    for j in range(TB):
        l = j * gtot + i
        bucket = sb_ref[l]
        tid = st_ref[l]

        q = q_refs[j][0, 0]
        qn = q * jax.lax.rsqrt(jnp.maximum(jnp.sum(q * q), 1e-24))
        anchor = cb_ref[pl.ds(bucket, 1), :]
        uq = ALPHA * qn + (1.0 - ALPHA) * anchor
        uq = uq * jax.lax.rsqrt(jnp.maximum(jnp.sum(uq * uq), 1e-24))

        keys = k_refs[j][0]
        vals = v_refs[j][0]
        scores = jax.lax.dot_general(
            uq, keys, (((1,), (1,)), ((), ())),
            preferred_element_type=jnp.float32)

        stids = st_refs[j][0, 0]
        mask = (stids == tid).astype(jnp.float32)
        msum = jnp.sum(mask)
        has_match = msum > 0.0

        probs_hard = mask / (msum + 1e-9)
        s2 = scores * (1.0 / TAU)
        e = jnp.exp(s2 - jnp.max(s2))
        probs_soft = e / jnp.sum(e)
        probs = jnp.where(has_match, probs_hard, probs_soft)

        val = jax.lax.dot_general(
            probs, vals, (((1,), (0,)), ((), ())),
            preferred_element_type=jnp.float32)
        out_refs[j][0, 0] = val[0]

        sim = jnp.where(has_match, 10.0, jnp.max(scores))
        sim_refs[j][0, 0] = jnp.full((128,), sim, dtype=jnp.float32)


@jax.jit
def kernel(query_emb, slot_values, slot_keys, tids, centroid_codebook,
           slot_tids):
    B, T, D = query_emb.shape
    gtot = (B * T) // TB
    buckets = tids % N_BUCKETS
    order = jnp.argsort(buckets, axis=-1)
    sb = jnp.take_along_axis(buckets, order, axis=-1).reshape(B * T)
    st = jnp.take_along_axis(tids, order, axis=-1).reshape(B * T)
    order_flat = order.reshape(B * T)
    stids4 = slot_tids.reshape(B, N_BUCKETS, 1, S)
    q4 = query_emb.reshape(B, T, 1, D)

    def q_map(j):
        def m(i, sbr, str_, orr):
            l = j * gtot + i
            return (l // T, orr[l], 0, 0)
        return m

    def kv_map(j):
        def m(i, sbr, str_, orr):
            l = j * gtot + i
            return (l // T, sbr[l], 0)
        return m

    def stid_map(j):
        def m(i, sbr, str_, orr):
            l = j * gtot + i
            return (l // T, sbr[l], 0, 0)
        return m

    def cb_map(i, sbr, str_, orr):
        return (0, 0)

    def sorted_out_map(i, sbr, str_, orr):
        return (i, 0, 0)

    in_specs = [pl.BlockSpec((1, 1, 1, D), q_map(j)) for j in range(TB)]
    in_specs += [pl.BlockSpec((1, S, D), kv_map(j)) for j in range(TB)]
    in_specs += [pl.BlockSpec((1, S, D), kv_map(j)) for j in range(TB)]
    in_specs += [pl.BlockSpec((1, 1, 1, S), stid_map(j)) for j in range(TB)]
    in_specs += [pl.BlockSpec((N_BUCKETS, D), cb_map)]

    out_specs = [pl.BlockSpec((1, 1, D), sorted_out_map) for _ in range(TB)]
    out_specs += [pl.BlockSpec((1, 1, 128), sorted_out_map)
                  for _ in range(TB)]

    grid_spec = pltpu.PrefetchScalarGridSpec(
        num_scalar_prefetch=3,
        grid=(gtot,),
        in_specs=in_specs,
        out_specs=out_specs,
    )

    args = ([sb, st, order_flat]
            + [q4] * TB + [slot_keys] * TB + [slot_values] * TB
            + [stids4] * TB + [centroid_codebook])
    outs = pl.pallas_call(
        _token_kernel,
        grid_spec=grid_spec,
        out_shape=([jax.ShapeDtypeStruct((gtot, 1, D), jnp.float32)] * TB
                   + [jax.ShapeDtypeStruct((gtot, 1, 128), jnp.float32)] * TB),
    )(*args)

    vals_sorted = jnp.concatenate(
        [o.reshape(gtot, D) for o in outs[:TB]], axis=0)
    sims_sorted = jnp.concatenate(
        [o[:, 0, 0] for o in outs[TB:]], axis=0)
    inv = jnp.argsort(order, axis=-1)
    l_idx = (inv + jnp.arange(B, dtype=inv.dtype)[:, None] * T).reshape(-1)
    out = vals_sorted[l_idx].reshape(B, T, D)
    sim = sims_sorted[l_idx].reshape(B, T)
    return out, sim


# speedup vs baseline: 5.8040x; 5.8040x over previous
import jax
import jax.numpy as jnp
from jax.experimental import pallas as pl
from jax.experimental.pallas import tpu as pltpu

N_BUCKETS = 512
S = 32
TAU = 0.1
ALPHA = 0.5
TB = 16


def _token_kernel(buckets_ref, tids_pref,
                  q_ref,
                  tid_ref,
                  *refs):
    k_refs = refs[0:TB]
    v_refs = refs[TB:2 * TB]
    st_refs = refs[2 * TB:3 * TB]
    cb_ref = refs[3 * TB]
    out_ref = refs[3 * TB + 1]
    sim_ref = refs[3 * TB + 2]

    i = pl.program_id(0)
    base = i * TB

    qs = q_ref[0, 0]
    qn = qs * jax.lax.rsqrt(
        jnp.maximum(jnp.sum(qs * qs, axis=1, keepdims=True), 1e-24))

    anchors = jnp.concatenate(
        [cb_ref[pl.ds(buckets_ref[base + j], 1), :] for j in range(TB)],
        axis=0)

    uq = ALPHA * qn + (1.0 - ALPHA) * anchors
    uq = uq * jax.lax.rsqrt(
        jnp.maximum(jnp.sum(uq * uq, axis=1, keepdims=True), 1e-24))

    scores = jnp.concatenate([
        jax.lax.dot_general(
            uq[j:j + 1, :], k_refs[j][0], (((1,), (1,)), ((), ())),
            preferred_element_type=jnp.float32)
        for j in range(TB)], axis=0)

    stids = jnp.concatenate([st_refs[j][0, 0] for j in range(TB)], axis=0)
    tid_col = tid_ref[0, 0]
    mask = (stids == tid_col).astype(jnp.float32)
    msum = jnp.sum(mask, axis=1, keepdims=True)
    has_match = msum > 0.0

    probs_hard = mask / (msum + 1e-9)
    s2 = scores * (1.0 / TAU)
    smax = jnp.max(s2, axis=1, keepdims=True)
    e = jnp.exp(s2 - smax)
    probs_soft = e / jnp.sum(e, axis=1, keepdims=True)
    probs = jnp.where(has_match, probs_hard, probs_soft)

    vals = jnp.concatenate([
        jax.lax.dot_general(
            probs[j:j + 1, :], v_refs[j][0], (((1,), (0,)), ((), ())),
            preferred_element_type=jnp.float32)
        for j in range(TB)], axis=0)
    out_ref[0, 0] = vals

    max_scores = jnp.max(scores, axis=1, keepdims=True)
    sim = jnp.where(has_match, 10.0, max_scores)
    sim_ref[0, 0] = sim * jnp.ones((1, 128), jnp.float32)


@jax.jit
def kernel(query_emb, slot_values, slot_keys, tids, centroid_codebook,
           slot_tids):
    B, T, D = query_emb.shape
    G = T // TB
    buckets = (tids % N_BUCKETS).reshape(B * T)
    tids_flat = tids.reshape(B * T)
    stids4 = slot_tids.reshape(B, N_BUCKETS, 1, S)
    q4 = query_emb.reshape(B, G, TB, D)
    tid4 = tids.reshape(B, G, TB, 1)

    grid = (B * G,)

    def q_map(i, bk, tf):
        return (i // G, i % G, 0, 0)

    def kv_map(j):
        def m(i, bk, tf):
            return (i // G, bk[i * TB + j], 0)
        return m

    def st_map(j):
        def m(i, bk, tf):
            return (i // G, bk[i * TB + j], 0, 0)
        return m

    def cb_map(i, bk, tf):
        return (0, 0)

    in_specs = [pl.BlockSpec((1, 1, TB, D), q_map),
                pl.BlockSpec((1, 1, TB, 1), q_map)]
    in_specs += [pl.BlockSpec((1, S, D), kv_map(j)) for j in range(TB)]
    in_specs += [pl.BlockSpec((1, S, D), kv_map(j)) for j in range(TB)]
    in_specs += [pl.BlockSpec((1, 1, 1, S), st_map(j)) for j in range(TB)]
    in_specs += [pl.BlockSpec((N_BUCKETS, D), cb_map)]

    grid_spec = pltpu.PrefetchScalarGridSpec(
        num_scalar_prefetch=2,
        grid=grid,
        in_specs=in_specs,
        out_specs=[
            pl.BlockSpec((1, 1, TB, D), q_map),
            pl.BlockSpec((1, 1, TB, 128), q_map),
        ],
    )

    args = ([buckets, tids_flat, q4, tid4]
            + [slot_keys] * TB + [slot_values] * TB + [stids4] * TB
            + [centroid_codebook])
    out, sim = pl.pallas_call(
        _token_kernel,
        grid_spec=grid_spec,
        out_shape=[
            jax.ShapeDtypeStruct((B, G, TB, D), jnp.float32),
            jax.ShapeDtypeStruct((B, G, TB, 128), jnp.float32),
        ],
    )(*args)
    return out.reshape(B, T, D), sim[:, :, :, 0].reshape(B, T)


# speedup vs baseline: 8.0339x; 1.3842x over previous
import functools

import jax
import jax.numpy as jnp
from jax.experimental import pallas as pl
from jax.experimental.pallas import tpu as pltpu

N_BUCKETS = 512
S = 32
TAU = 0.1
ALPHA = 0.5
TB = 32


def _token_kernel(g_per_row,
                  buckets_ref, tids_pref,
                  q_ref,
                  tid_ref,
                  *refs):
    k_refs = refs[0:TB]
    v_refs = refs[TB:2 * TB]
    cb_ref = refs[2 * TB]
    stid_ref = refs[2 * TB + 1]
    out_ref = refs[2 * TB + 2]
    sim_ref = refs[2 * TB + 3]

    i = pl.program_id(0)
    base = i * TB
    b = i // g_per_row

    qs = q_ref[0, 0]
    qn = qs * jax.lax.rsqrt(
        jnp.maximum(jnp.sum(qs * qs, axis=1, keepdims=True), 1e-24))

    anchors = jnp.concatenate(
        [cb_ref[pl.ds(buckets_ref[base + j], 1), :] for j in range(TB)],
        axis=0)

    uq = ALPHA * qn + (1.0 - ALPHA) * anchors
    uq = uq * jax.lax.rsqrt(
        jnp.maximum(jnp.sum(uq * uq, axis=1, keepdims=True), 1e-24))

    scores = jnp.concatenate([
        jax.lax.dot_general(
            uq[j:j + 1, :], k_refs[j][0], (((1,), (1,)), ((), ())),
            preferred_element_type=jnp.float32)
        for j in range(TB)], axis=0)

    row0 = b * N_BUCKETS
    stids = jnp.concatenate(
        [stid_ref[pl.ds(row0 + buckets_ref[base + j], 1), :]
         for j in range(TB)], axis=0)
    tid_col = tid_ref[0, 0]
    mask = (stids == tid_col).astype(jnp.float32)
    msum = jnp.sum(mask, axis=1, keepdims=True)
    has_match = msum > 0.0

    probs_hard = mask / (msum + 1e-9)
    s2 = scores * (1.0 / TAU)
    smax = jnp.max(s2, axis=1, keepdims=True)
    e = jnp.exp(s2 - smax)
    probs_soft = e / jnp.sum(e, axis=1, keepdims=True)
    probs = jnp.where(has_match, probs_hard, probs_soft)

    vals = jnp.concatenate([
        jax.lax.dot_general(
            probs[j:j + 1, :], v_refs[j][0], (((1,), (0,)), ((), ())),
            preferred_element_type=jnp.float32)
        for j in range(TB)], axis=0)
    out_ref[0, 0] = vals

    max_scores = jnp.max(scores, axis=1, keepdims=True)
    sim = jnp.where(has_match, 10.0, max_scores)
    sim_ref[0, 0] = sim * jnp.ones((1, 128), jnp.float32)


@jax.jit
def kernel(query_emb, slot_values, slot_keys, tids, centroid_codebook,
           slot_tids):
    B, T, D = query_emb.shape
    G = T // TB
    buckets = (tids % N_BUCKETS).reshape(B * T)
    tids_flat = tids.reshape(B * T)
    stid_tab = slot_tids.reshape(B * N_BUCKETS, S)
    q4 = query_emb.reshape(B, G, TB, D)
    tid4 = tids.reshape(B, G, TB, 1)

    grid = (B * G,)

    def q_map(i, bk, tf):
        return (i // G, i % G, 0, 0)

    def kv_map(j):
        def m(i, bk, tf):
            return (i // G, bk[i * TB + j], 0)
        return m

    def cb_map(i, bk, tf):
        return (0, 0)

    in_specs = [pl.BlockSpec((1, 1, TB, D), q_map),
                pl.BlockSpec((1, 1, TB, 1), q_map)]
    in_specs += [pl.BlockSpec((1, S, D), kv_map(j)) for j in range(TB)]
    in_specs += [pl.BlockSpec((1, S, D), kv_map(j)) for j in range(TB)]
    in_specs += [pl.BlockSpec((N_BUCKETS, D), cb_map),
                 pl.BlockSpec((B * N_BUCKETS, S), cb_map)]

    grid_spec = pltpu.PrefetchScalarGridSpec(
        num_scalar_prefetch=2,
        grid=grid,
        in_specs=in_specs,
        out_specs=[
            pl.BlockSpec((1, 1, TB, D), q_map),
            pl.BlockSpec((1, 1, TB, 128), q_map),
        ],
    )

    args = ([buckets, tids_flat, q4, tid4]
            + [slot_keys] * TB + [slot_values] * TB
            + [centroid_codebook, stid_tab])
    out, sim = pl.pallas_call(
        functools.partial(_token_kernel, G),
        grid_spec=grid_spec,
        out_shape=[
            jax.ShapeDtypeStruct((B, G, TB, D), jnp.float32),
            jax.ShapeDtypeStruct((B, G, TB, 128), jnp.float32),
        ],
    )(*args)
    return out.reshape(B, T, D), sim[:, :, :, 0].reshape(B, T)
